# baseline (speedup 1.0000x reference)
import functools
import jax
import jax.numpy as jnp
from jax import lax
from jax.experimental import pallas as pl
from jax.experimental.pallas import tpu as pltpu
from jax.experimental.pallas import tpu_sc as plsc

_B, _T, _P, _H = 8, 4, 1025, 1280
_LG = _H // 16
_CROWS = 8
_NFULL = _P // _CROWS


def _sc_body(hid, ids, table, g16, out, idx_v, emb_v, g_v, buf, tail, gsem):
    c = lax.axis_index("c")
    s = lax.axis_index("s")
    wid = s * 2 + c
    b = wid // _T
    t = wid % _T

    pltpu.sync_copy(ids, idx_v)
    pltpu.sync_copy(g16, g_v)
    pltpu.async_copy(table.at[idx_v], emb_v, gsem).wait()

    g = g_v[pl.ds(0, 16)]
    e2 = jnp.exp(g * 2.0)
    scale = 1.0 - 2.0 / (e2 + 1.0)

    def chunk_body(k, carry):
        r0 = k * _CROWS
        pltpu.sync_copy(hid.at[b, t, pl.ds(r0, _CROWS), :], buf)
        for lg in range(_LG):
            av = emb_v[b, pl.ds(t * _H + lg * 16, 16)] * scale
            for rr in range(_CROWS):
                buf[rr, pl.ds(lg * 16, 16)] = buf[rr, pl.ds(lg * 16, 16)] + av
        pltpu.sync_copy(buf, out.at[b, t, pl.ds(r0, _CROWS), :])
        return carry

    lax.fori_loop(0, _NFULL, chunk_body, 0)

    r0 = _NFULL * _CROWS
    pltpu.sync_copy(hid.at[b, t, pl.ds(r0, 1), :], tail)
    for lg in range(_LG):
        av = emb_v[b, pl.ds(t * _H + lg * 16, 16)] * scale
        tail[0, pl.ds(lg * 16, 16)] = tail[0, pl.ds(lg * 16, 16)] + av
    pltpu.sync_copy(tail, out.at[b, t, pl.ds(r0, 1), :])


def kernel(hidden_state, aspect_ratio_ids, embedding_table, gate):
    g16 = jnp.broadcast_to(gate, (16,))
    sc_kernel = pl.kernel(
        _sc_body,
        out_type=jax.ShapeDtypeStruct(hidden_state.shape, hidden_state.dtype),
        mesh=plsc.VectorSubcoreMesh(core_axis_name="c", subcore_axis_name="s"),
        scratch_types=[
            pltpu.VMEM((8,), jnp.int32),
            pltpu.VMEM((_B, _T * _H), jnp.float32),
            pltpu.VMEM((16,), jnp.float32),
            pltpu.VMEM((_CROWS, _H), jnp.float32),
            pltpu.VMEM((1, _H), jnp.float32),
            pltpu.SemaphoreType.DMA,
        ],
        compiler_params=pltpu.CompilerParams(use_tc_tiling_on_sc=True),
    )
    return sc_kernel(hidden_state, aspect_ratio_ids, embedding_table, g16)


# speedup vs baseline: 1.3742x; 1.3742x over previous
import functools
import jax
import jax.numpy as jnp
from jax import lax
from jax.experimental import pallas as pl
from jax.experimental.pallas import tpu as pltpu
from jax.experimental.pallas import tpu_sc as plsc

_B, _T, _P, _H = 8, 4, 1025, 1280
_LG = _H // 16
_CR = 16
_NF = 1024 // _CR


def _sc_body(hid, ids, table, g16, out,
             idx_v, emb_v, g_v, my_emb, in0, in1, out0, out1,
             isem, osem, gsem):
    c = lax.axis_index("c")
    s = lax.axis_index("s")
    wid = s * 2 + c
    b = wid // _T
    t = wid % _T

    pltpu.sync_copy(ids, idx_v)
    pltpu.sync_copy(g16, g_v)
    pltpu.async_copy(table.at[idx_v], emb_v, gsem).wait()

    g = g_v[pl.ds(0, 16)]
    scale = 1.0 - 2.0 / (jnp.exp(g * 2.0) + 1.0)

    def scale_body(lg, carry):
        off = lg * 16
        my_emb[0, pl.ds(off, 16)] = (
            emb_v[b, pl.ds(t * _H + off, 16)] * scale)
        return carry

    lax.fori_loop(0, _LG, scale_body, 0)

    ins = (in0, in1)
    outs = (out0, out1)

    def icopy(k, slot):
        return pltpu.make_async_copy(
            hid.at[b, t, pl.ds(k * _CR, _CR), :], ins[slot], isem.at[slot])

    def ocopy(k, slot):
        return pltpu.make_async_copy(
            outs[slot], out.at[b, t, pl.ds(k * _CR, _CR), :], osem.at[slot])

    def compute(slot):
        ib, ob = ins[slot], outs[slot]

        def lg_body(lg, carry):
            off = lg * 16
            av = my_emb[0, pl.ds(off, 16)]
            for rr in range(_CR):
                ob[rr, pl.ds(off, 16)] = ib[rr, pl.ds(off, 16)] + av
            return carry

        lax.fori_loop(0, _LG, lg_body, 0)

    icopy(0, 0).start()
    icopy(1, 1).start()
    icopy(0, 0).wait()
    compute(0)
    ocopy(0, 0).start()
    icopy(2, 0).start()
    icopy(1, 1).wait()
    compute(1)
    ocopy(1, 1).start()
    icopy(3, 1).start()

    def body(gi, carry):
        k0 = gi * 2
        k1 = k0 + 1
        icopy(k0, 0).wait()
        ocopy(k0 - 2, 0).wait()
        compute(0)
        ocopy(k0, 0).start()
        icopy(k0 + 2, 0).start()
        icopy(k1, 1).wait()
        ocopy(k1 - 2, 1).wait()
        compute(1)
        ocopy(k1, 1).start()
        icopy(k1 + 2, 1).start()
        return carry

    lax.fori_loop(1, _NF // 2 - 1, body, 0)

    icopy(_NF - 2, 0).wait()
    ocopy(_NF - 4, 0).wait()
    compute(0)
    ocopy(_NF - 2, 0).start()
    icopy(_NF - 1, 1).wait()
    ocopy(_NF - 3, 1).wait()
    compute(1)
    ocopy(_NF - 1, 1).start()
    ocopy(_NF - 2, 0).wait()
    ocopy(_NF - 1, 1).wait()

    r0 = _NF * _CR
    pltpu.sync_copy(hid.at[b, t, pl.ds(r0, 1), :], in0.at[pl.ds(0, 1), :])

    def tail_body(lg, carry):
        off = lg * 16
        av = my_emb[0, pl.ds(off, 16)]
        out0[0, pl.ds(off, 16)] = in0[0, pl.ds(off, 16)] + av
        return carry

    lax.fori_loop(0, _LG, tail_body, 0)
    pltpu.sync_copy(out0.at[pl.ds(0, 1), :], out.at[b, t, pl.ds(r0, 1), :])


def kernel(hidden_state, aspect_ratio_ids, embedding_table, gate):
    g16 = jnp.broadcast_to(gate, (16,))
    sc_kernel = pl.kernel(
        _sc_body,
        out_type=jax.ShapeDtypeStruct(hidden_state.shape, hidden_state.dtype),
        mesh=plsc.VectorSubcoreMesh(core_axis_name="c", subcore_axis_name="s"),
        scratch_types=[
            pltpu.VMEM((8,), jnp.int32),
            pltpu.VMEM((_B, _T * _H), jnp.float32),
            pltpu.VMEM((16,), jnp.float32),
            pltpu.VMEM((1, _H), jnp.float32),
            pltpu.VMEM((_CR, _H), jnp.float32),
            pltpu.VMEM((_CR, _H), jnp.float32),
            pltpu.VMEM((_CR, _H), jnp.float32),
            pltpu.VMEM((_CR, _H), jnp.float32),
            pltpu.SemaphoreType.DMA((2,)),
            pltpu.SemaphoreType.DMA((2,)),
            pltpu.SemaphoreType.DMA,
        ],
        compiler_params=pltpu.CompilerParams(use_tc_tiling_on_sc=True),
    )
    return sc_kernel(hidden_state, aspect_ratio_ids, embedding_table, g16)


# speedup vs baseline: 1.3960x; 1.0159x over previous
import functools
import jax
import jax.numpy as jnp
from jax import lax
from jax.experimental import pallas as pl
from jax.experimental.pallas import tpu as pltpu
from jax.experimental.pallas import tpu_sc as plsc

_B, _T, _P, _H = 8, 4, 1025, 1280
_LG = _H // 16
_CR = 24
_NF = _P // _CR
_TAIL = _P - _NF * _CR


def _sc_body(hid, idx8, table, g16, out,
             idx_v, g_v, my_emb, in0, in1, out0, out1,
             isem, osem, gsem):
    c = lax.axis_index("c")
    s = lax.axis_index("s")
    wid = s * 2 + c
    b = wid // _T
    t = wid % _T

    pltpu.sync_copy(idx8.at[b], idx_v)
    pltpu.async_copy(table.at[idx_v], out0.at[pl.ds(0, 8), :], gsem).wait()

    pltpu.sync_copy(g16, g_v)
    g = g_v[pl.ds(0, 16)]
    scale = 1.0 - 2.0 / (jnp.exp(g * 2.0) + 1.0)

    def scale_body(lg, carry):
        off = lg * 16
        my_emb[0, pl.ds(off, 16)] = out0[t, pl.ds(off, 16)] * scale
        return carry

    lax.fori_loop(0, _LG, scale_body, 0)

    ins = (in0, in1)
    outs = (out0, out1)

    def icopy(k, slot):
        return pltpu.make_async_copy(
            hid.at[b, t, pl.ds(k * _CR, _CR), :], ins[slot], isem.at[slot])

    def ocopy(k, slot):
        return pltpu.make_async_copy(
            outs[slot], out.at[b, t, pl.ds(k * _CR, _CR), :], osem.at[slot])

    def compute(slot):
        ib, ob = ins[slot], outs[slot]

        def lg_body(lg, carry):
            off = lg * 16
            av = my_emb[0, pl.ds(off, 16)]
            for rr in range(_CR):
                ob[rr, pl.ds(off, 16)] = ib[rr, pl.ds(off, 16)] + av
            return carry

        lax.fori_loop(0, _LG, lg_body, 0)

    icopy(0, 0).start()
    icopy(1, 1).start()
    icopy(0, 0).wait()
    compute(0)
    ocopy(0, 0).start()
    icopy(2, 0).start()
    icopy(1, 1).wait()
    compute(1)
    ocopy(1, 1).start()
    icopy(3, 1).start()

    def body(gi, carry):
        k0 = gi * 2
        k1 = k0 + 1
        icopy(k0, 0).wait()
        ocopy(k0 - 2, 0).wait()
        compute(0)
        ocopy(k0, 0).start()
        icopy(k0 + 2, 0).start()
        icopy(k1, 1).wait()
        ocopy(k1 - 2, 1).wait()
        compute(1)
        ocopy(k1, 1).start()
        icopy(k1 + 2, 1).start()
        return carry

    lax.fori_loop(1, _NF // 2 - 1, body, 0)

    icopy(_NF - 2, 0).wait()
    ocopy(_NF - 4, 0).wait()
    compute(0)
    ocopy(_NF - 2, 0).start()
    icopy(_NF - 1, 1).wait()
    ocopy(_NF - 3, 1).wait()
    compute(1)
    ocopy(_NF - 1, 1).start()
    ocopy(_NF - 2, 0).wait()
    ocopy(_NF - 1, 1).wait()

    r0 = _NF * _CR
    pltpu.sync_copy(hid.at[b, t, pl.ds(r0, 16), :], in0.at[pl.ds(0, 16), :])

    def tail16_body(lg, carry):
        off = lg * 16
        av = my_emb[0, pl.ds(off, 16)]
        for rr in range(16):
            out0[rr, pl.ds(off, 16)] = in0[rr, pl.ds(off, 16)] + av
        return carry

    lax.fori_loop(0, _LG, tail16_body, 0)
    pltpu.sync_copy(out0.at[pl.ds(0, 16), :], out.at[b, t, pl.ds(r0, 16), :])

    r1 = r0 + 16
    pltpu.sync_copy(hid.at[b, t, pl.ds(r1, 1), :], in1.at[pl.ds(0, 1), :])

    def tail1_body(lg, carry):
        off = lg * 16
        av = my_emb[0, pl.ds(off, 16)]
        out1[0, pl.ds(off, 16)] = in1[0, pl.ds(off, 16)] + av
        return carry

    lax.fori_loop(0, _LG, tail1_body, 0)
    pltpu.sync_copy(out1.at[pl.ds(0, 1), :], out.at[b, t, pl.ds(r1, 1), :])


def kernel(hidden_state, aspect_ratio_ids, embedding_table, gate):
    g16 = jnp.broadcast_to(gate, (16,))
    tiles8 = jnp.tile(jnp.arange(_T, dtype=jnp.int32), 2)
    idx8 = aspect_ratio_ids.astype(jnp.int32)[:, None] * _T + tiles8[None, :]
    table36 = embedding_table.reshape(-1, _H)
    sc_kernel = pl.kernel(
        _sc_body,
        out_type=jax.ShapeDtypeStruct(hidden_state.shape, hidden_state.dtype),
        mesh=plsc.VectorSubcoreMesh(core_axis_name="c", subcore_axis_name="s"),
        scratch_types=[
            pltpu.VMEM((8,), jnp.int32),
            pltpu.VMEM((16,), jnp.float32),
            pltpu.VMEM((1, _H), jnp.float32),
            pltpu.VMEM((_CR, _H), jnp.float32),
            pltpu.VMEM((_CR, _H), jnp.float32),
            pltpu.VMEM((_CR, _H), jnp.float32),
            pltpu.VMEM((_CR, _H), jnp.float32),
            pltpu.SemaphoreType.DMA((2,)),
            pltpu.SemaphoreType.DMA((2,)),
            pltpu.SemaphoreType.DMA,
        ],
        compiler_params=pltpu.CompilerParams(use_tc_tiling_on_sc=True),
    )
    return sc_kernel(hidden_state, idx8, table36, g16)


# speedup vs baseline: 1.4052x; 1.0066x over previous
import functools
import jax
import jax.numpy as jnp
from jax import lax
from jax.experimental import pallas as pl
from jax.experimental.pallas import tpu as pltpu
from jax.experimental.pallas import tpu_sc as plsc

_B, _T, _P, _H = 8, 4, 1025, 1280
_LG = _H // 16
_CR = 16
_NF = 1024 // _CR
_R = 3


def _sc_body(hid, idx8, table, g16, out,
             idx_v, g_v, my_emb, in0, in1, in2, out0, out1, out2,
             isem, osem, gsem):
    c = lax.axis_index("c")
    s = lax.axis_index("s")
    wid = s * 2 + c
    b = wid // _T
    t = wid % _T

    pltpu.sync_copy(idx8.at[b], idx_v)
    pltpu.async_copy(table.at[idx_v], out0.at[pl.ds(0, 8), :], gsem).wait()

    pltpu.sync_copy(g16, g_v)
    g = g_v[pl.ds(0, 16)]
    scale = 1.0 - 2.0 / (jnp.exp(g * 2.0) + 1.0)

    def scale_body(lg, carry):
        off = lg * 16
        my_emb[0, pl.ds(off, 16)] = out0[t, pl.ds(off, 16)] * scale
        return carry

    lax.fori_loop(0, _LG, scale_body, 0)

    ins = (in0, in1, in2)
    outs = (out0, out1, out2)

    def icopy(k, slot):
        return pltpu.make_async_copy(
            hid.at[b, t, pl.ds(k * _CR, _CR), :], ins[slot], isem.at[slot])

    def ocopy(k, slot):
        return pltpu.make_async_copy(
            outs[slot], out.at[b, t, pl.ds(k * _CR, _CR), :], osem.at[slot])

    def compute(slot):
        ib, ob = ins[slot], outs[slot]

        def lg_body(lg, carry):
            off = lg * 16
            av = my_emb[0, pl.ds(off, 16)]
            for rr in range(_CR):
                ob[rr, pl.ds(off, 16)] = ib[rr, pl.ds(off, 16)] + av
            return carry

        lax.fori_loop(0, _LG, lg_body, 0)

    for j in range(_R):
        icopy(j, j).start()
    for j in range(_R):
        icopy(j, j).wait()
        compute(j)
        ocopy(j, j).start()
        icopy(j + _R, j).start()

    def body(gi, carry):
        k0 = gi * _R
        for j in range(_R):
            k = k0 + j
            icopy(k, j).wait()
            ocopy(k - _R, j).wait()
            compute(j)
            ocopy(k, j).start()
            icopy(k + _R, j).start()
        return carry

    lax.fori_loop(1, _NF // _R - 1, body, 0)

    k0 = (_NF // _R - 1) * _R
    for k in range(k0, _NF):
        j = k % _R
        icopy(k, j).wait()
        ocopy(k - _R, j).wait()
        compute(j)
        ocopy(k, j).start()
        if k + _R < _NF:
            icopy(k + _R, j).start()
    for k in range(_NF - _R, _NF):
        ocopy(k, k % _R).wait()

    r1 = _NF * _CR
    pltpu.sync_copy(hid.at[b, t, pl.ds(r1, 1), :], in1.at[pl.ds(0, 1), :])

    def tail1_body(lg, carry):
        off = lg * 16
        av = my_emb[0, pl.ds(off, 16)]
        out1[0, pl.ds(off, 16)] = in1[0, pl.ds(off, 16)] + av
        return carry

    lax.fori_loop(0, _LG, tail1_body, 0)
    pltpu.sync_copy(out1.at[pl.ds(0, 1), :], out.at[b, t, pl.ds(r1, 1), :])


def kernel(hidden_state, aspect_ratio_ids, embedding_table, gate):
    g16 = jnp.broadcast_to(gate, (16,))
    tiles8 = jnp.tile(jnp.arange(_T, dtype=jnp.int32), 2)
    idx8 = aspect_ratio_ids.astype(jnp.int32)[:, None] * _T + tiles8[None, :]
    table36 = embedding_table.reshape(-1, _H)
    sc_kernel = pl.kernel(
        _sc_body,
        out_type=jax.ShapeDtypeStruct(hidden_state.shape, hidden_state.dtype),
        mesh=plsc.VectorSubcoreMesh(core_axis_name="c", subcore_axis_name="s"),
        scratch_types=[
            pltpu.VMEM((8,), jnp.int32),
            pltpu.VMEM((16,), jnp.float32),
            pltpu.VMEM((1, _H), jnp.float32),
            pltpu.VMEM((_CR, _H), jnp.float32),
            pltpu.VMEM((_CR, _H), jnp.float32),
            pltpu.VMEM((_CR, _H), jnp.float32),
            pltpu.VMEM((_CR, _H), jnp.float32),
            pltpu.VMEM((_CR, _H), jnp.float32),
            pltpu.VMEM((_CR, _H), jnp.float32),
            pltpu.SemaphoreType.DMA((_R,)),
            pltpu.SemaphoreType.DMA((_R,)),
            pltpu.SemaphoreType.DMA,
        ],
        compiler_params=pltpu.CompilerParams(use_tc_tiling_on_sc=True),
    )
    return sc_kernel(hidden_state, idx8, table36, g16)


# speedup vs baseline: 1.4052x; 1.0000x over previous
import jax
import jax.numpy as jnp
from jax import lax
from jax.experimental import pallas as pl
from jax.experimental.pallas import tpu as pltpu
from jax.experimental.pallas import tpu_sc as plsc

_B, _T, _P, _H = 8, 4, 1025, 1280
_LG = _H // 16
_CR = 16
_NF = 1024 // _CR
_R = 3


def _sc_body(hid, idx8, table, g16, out,
             idx_v, g_v, my_emb, in0, in1, in2, out0, out1, out2,
             isem, osem, gsem):
    c = lax.axis_index("c")
    s = lax.axis_index("s")
    wid = s * 2 + c
    b = wid // _T
    t = wid % _T

    pltpu.sync_copy(idx8.at[b], idx_v)
    pltpu.async_copy(table.at[idx_v], out0.at[pl.ds(0, 8), :], gsem).wait()

    pltpu.sync_copy(g16, g_v)
    g = g_v[pl.ds(0, 16)]
    scale = 1.0 - 2.0 / (jnp.exp(g * 2.0) + 1.0)

    def scale_body(lg, carry):
        off = lg * 16
        my_emb[0, pl.ds(off, 16)] = out0[t, pl.ds(off, 16)] * scale
        return carry

    lax.fori_loop(0, _LG, scale_body, 0)

    ins = (in0, in1, in2)
    outs = (out0, out1, out2)

    def icopy(k, slot):
        return pltpu.make_async_copy(
            hid.at[b, t, pl.ds(k * _CR, _CR), :], ins[slot], isem.at[slot])

    def ocopy(k, slot):
        return pltpu.make_async_copy(
            outs[slot], out.at[b, t, pl.ds(k * _CR, _CR), :], osem.at[slot])

    def compute(slot):
        ib, ob = ins[slot], outs[slot]

        def lg_body(lg, carry):
            off = lg * 16
            av = my_emb[0, pl.ds(off, 16)]
            for rr in range(_CR):
                ob[rr, pl.ds(off, 16)] = ib[rr, pl.ds(off, 16)] + av
            return carry

        lax.fori_loop(0, _LG, lg_body, 0)

    for j in range(_R):
        icopy(j, j).start()
    for j in range(_R):
        icopy(j, j).wait()
        compute(j)
        ocopy(j, j).start()
        icopy(j + _R, j).start()

    def body(gi, carry):
        k0 = gi * _R
        for j in range(_R):
            k = k0 + j
            icopy(k, j).wait()
            ocopy(k - _R, j).wait()
            compute(j)
            ocopy(k, j).start()
            icopy(k + _R, j).start()
        return carry

    lax.fori_loop(1, _NF // _R - 1, body, 0)

    k0 = (_NF // _R - 1) * _R
    for k in range(k0, _NF):
        j = k % _R
        icopy(k, j).wait()
        ocopy(k - _R, j).wait()
        compute(j)
        ocopy(k, j).start()
        if k + _R < _NF:
            icopy(k + _R, j).start()
    for k in range(_NF - _R, _NF):
        ocopy(k, k % _R).wait()

    r1 = _NF * _CR
    pltpu.sync_copy(hid.at[b, t, pl.ds(r1, 1), :], in1.at[pl.ds(0, 1), :])

    def tail1_body(lg, carry):
        off = lg * 16
        av = my_emb[0, pl.ds(off, 16)]
        out1[0, pl.ds(off, 16)] = in1[0, pl.ds(off, 16)] + av
        return carry

    lax.fori_loop(0, _LG, tail1_body, 0)
    pltpu.sync_copy(out1.at[pl.ds(0, 1), :], out.at[b, t, pl.ds(r1, 1), :])


def kernel(hidden_state, aspect_ratio_ids, embedding_table, gate):
    g16 = jnp.broadcast_to(gate, (16,))
    tiles8 = jnp.tile(jnp.arange(_T, dtype=jnp.int32), 2)
    idx8 = aspect_ratio_ids.astype(jnp.int32)[:, None] * _T + tiles8[None, :]
    table36 = embedding_table.reshape(-1, _H)
    sc_kernel = pl.kernel(
        _sc_body,
        out_type=jax.ShapeDtypeStruct(hidden_state.shape, hidden_state.dtype),
        mesh=plsc.VectorSubcoreMesh(core_axis_name="c", subcore_axis_name="s"),
        scratch_types=[
            pltpu.VMEM((8,), jnp.int32),
            pltpu.VMEM((16,), jnp.float32),
            pltpu.VMEM((1, _H), jnp.float32),
            pltpu.VMEM((_CR, _H), jnp.float32),
            pltpu.VMEM((_CR, _H), jnp.float32),
            pltpu.VMEM((_CR, _H), jnp.float32),
            pltpu.VMEM((_CR, _H), jnp.float32),
            pltpu.VMEM((_CR, _H), jnp.float32),
            pltpu.VMEM((_CR, _H), jnp.float32),
            pltpu.SemaphoreType.DMA((_R,)),
            pltpu.SemaphoreType.DMA((_R,)),
            pltpu.SemaphoreType.DMA,
        ],
        compiler_params=pltpu.CompilerParams(use_tc_tiling_on_sc=True),
    )
    return sc_kernel(hidden_state, idx8, table36, g16)
